# Initial kernel scaffold; baseline (speedup 1.0000x reference)
import jax
import jax.numpy as jnp
from jax.experimental import pallas as pl


def kernel(x_employee, x_department, x_title, src_works_in, dst_works_in, src_has_role, dst_has_role, src_rev_works_in, dst_rev_works_in, src_rev_has_role, dst_rev_has_role, params):
    raise NotImplementedError("write your pallas kernel here")


# speedup vs baseline: 1.0001x; 1.0001x over previous
import jax
import jax.numpy as jnp
from jax.experimental import pallas as pl

NE, ND, NT = 100000, 500, 2000
E = 100000
DIN = 128
HID = 64
HEADS = 4
DOUT = 16


def _gat(x_src, x_dst, src, dst, p, heads, out_ch, n_dst):
    xs = (x_src @ p['W_src']).reshape(-1, heads, out_ch)
    xd = (x_dst @ p['W_dst']).reshape(-1, heads, out_ch)
    a_s = (xs * p['att_src'][None]).sum(-1)
    a_d = (xd * p['att_dst'][None]).sum(-1)
    alpha = jax.nn.leaky_relu(a_s[src] + a_d[dst], 0.2)
    amax = jax.ops.segment_max(alpha, dst, num_segments=n_dst)
    amax = jnp.where(jnp.isfinite(amax), amax, 0.0)
    ex = jnp.exp(alpha - amax[dst])
    den = jax.ops.segment_sum(ex, dst, num_segments=n_dst)
    attn = ex / (den[dst] + 1e-16)
    msg = xs[src] * attn[:, :, None]
    out = jax.ops.segment_sum(msg, dst, num_segments=n_dst)
    return out.reshape(n_dst, heads * out_ch) + p['bias']


def kernel(x_employee, x_department, x_title, src_works_in, dst_works_in, src_has_role, dst_has_role, src_rev_works_in, dst_rev_works_in, src_rev_has_role, dst_rev_has_role, params):
    x_e, x_d, x_t = x_employee, x_department, x_title
    swi, dwi, shr, dhr = src_works_in, dst_works_in, src_has_role, dst_has_role
    srwi, drwi, srhr, drhr = src_rev_works_in, dst_rev_works_in, src_rev_has_role, dst_rev_has_role
    h_d = _gat(x_e, x_d, swi, dwi, params['c1_wi'], HEADS, HID, ND)
    h_t = _gat(x_e, x_t, shr, dhr, params['c1_hr'], HEADS, HID, NT)
    h_e = _gat(x_d, x_e, srwi, drwi, params['c1_rwi'], HEADS, HID, NE) \
        + _gat(x_t, x_e, srhr, drhr, params['c1_rhr'], HEADS, HID, NE)
    h_e, h_d, h_t = jax.nn.relu(h_e), jax.nn.relu(h_d), jax.nn.relu(h_t)
    g_t = _gat(h_e, h_t, shr, dhr, params['c2_hr'], 1, HID, NT)
    g_d = _gat(h_e, h_d, swi, dwi, params['c2_wi'], 1, HID, ND)
    g_e = _gat(h_d, h_e, srwi, drwi, params['c2_rwi'], 1, HID, NE) \
        + _gat(h_t, h_e, srhr, drhr, params['c2_rhr'], 1, HID, NE)
    g_e = jax.nn.relu(g_e)
    out = g_e @ params['lin_W'] + params['lin_b']
    return jax.nn.log_softmax(out, axis=1)


# speedup vs baseline: 8.4565x; 8.4555x over previous
import functools

import jax
import jax.numpy as jnp
from jax import lax
from jax.experimental import pallas as pl
from jax.experimental.pallas import tpu as pltpu
from jax.experimental.pallas import tpu_sc as plsc

_NE, _ND, _NT = 100000, 500, 2000
_E = 100000
_DIN = 128
_HID = 64
_HEADS = 4
_DOUT = 16
_HP = 16
_EPAD = 106496
_B = 256
_BG = 128
_PADDST = 1 << 20


def _ru(x, m):
    return -(-x // m) * m


def _mesh():
    return plsc.VectorSubcoreMesh(core_axis_name="c", subcore_axis_name="s")


def _zero_fill(buf, rows, width16):
    z = jnp.zeros((16,), jnp.float32)

    def zb(r, _):
        for k in range(width16):
            buf[r, pl.ds(k * 16, 16)] = z
        return 0
    lax.fori_loop(0, rows, zb, 0)


@functools.lru_cache(None)
def _build_k1(D):
    Dh = _ru(D, 16) // 2
    JUNK = Dh
    DLr = _ru(Dh + 1, 128)
    ZT = DLr // 16
    ZB = min(128, ZT)
    NZ = -(-ZT // ZB)
    EPT = _EPAD // 16
    NB = EPT // _B
    WT = _ru(-(-Dh // 16), 8)

    @functools.partial(
        pl.kernel,
        out_type=(jax.ShapeDtypeStruct((_EPAD, _HP), jnp.float32),
                  jax.ShapeDtypeStruct((2 * Dh, _HP), jnp.float32)),
        mesh=_mesh(),
        compiler_params=pltpu.CompilerParams(use_tc_tiling_on_sc=False, needs_layout_passes=False),
        scratch_types=[
            pltpu.VMEM((_B,), jnp.int32),
            pltpu.VMEM((_B,), jnp.int32),
            pltpu.VMEM((2, 128), jnp.int32),
            pltpu.VMEM((2, 128), jnp.int32),
            pltpu.VMEM((2, 128), jnp.int32),
            pltpu.VMEM((_B, _HP), jnp.float32),
            pltpu.VMEM((_B, _HP), jnp.float32),
            pltpu.VMEM((_B, _HP), jnp.float32),
            pltpu.VMEM((ZB, _HP), jnp.float32),
            pltpu.VMEM_SHARED((DLr, _HP), jnp.float32),
            pltpu.SemaphoreType.DMA,
            pltpu.SemaphoreType.DMA,
        ],
    )
    def k1(a_s, a_d, src, dst, s_out, den_out,
           src_v, dst_v, src2, dstc2, ldst2, as_r, ad_r, s_r, zer, den_sh,
           sem1, sem2):
        cid = lax.axis_index("c")
        sid = lax.axis_index("s")
        lo = cid * Dh

        _zero_fill(zer, ZB, 1)
        zbase = sid * ZT
        for k in range(NZ):
            st = min(k * ZB, ZT - ZB)
            pltpu.sync_copy(zer, den_sh.at[pl.ds(zbase + st, ZB)])
        plsc.subcore_barrier()

        def step(j, carry):
            off = sid * EPT + j * _B
            pltpu.sync_copy(src.at[pl.ds(off, _B)], src_v)
            pltpu.sync_copy(dst.at[pl.ds(off, _B)], dst_v)
            for g in range(_B // 16):
                sv = src_v[pl.ds(g * 16, 16)]
                dv = dst_v[pl.ds(g * 16, 16)]
                dc = jnp.minimum(dv, D - 1)
                inh = (dv >= lo) & (dv < lo + Dh)
                ld = jnp.where(inh, dv - lo, JUNK)
                src2[g // 8, pl.ds((g % 8) * 16, 16)] = sv
                dstc2[g // 8, pl.ds((g % 8) * 16, 16)] = dc
                ldst2[g // 8, pl.ds((g % 8) * 16, 16)] = ld
            cps = []
            for k in range(2):
                cps.append(pltpu.async_copy(
                    a_s.at[src2.at[k]], as_r.at[pl.ds(k * 128, 128)], sem1))
                cps.append(pltpu.async_copy(
                    a_d.at[dstc2.at[k]], ad_r.at[pl.ds(k * 128, 128)], sem2))
            for cp in cps:
                cp.wait()

            def ebody(e4, _):
                for u in range(4):
                    e = e4 * 4 + u
                    x = as_r[e, :] + ad_r[e, :]
                    x = jnp.maximum(x, 0.2 * x)
                    s_r[e, :] = jnp.exp(x)
                return 0
            lax.fori_loop(0, _B // 4, ebody, 0)

            for k in range(2):
                pltpu.sync_copy(s_r.at[pl.ds(k * 128, 128)],
                                den_sh.at[ldst2.at[k]], add=True)

            @pl.when(cid == 0)
            def _():
                pltpu.sync_copy(s_r, s_out.at[pl.ds(off, _B)])
            return carry

        lax.fori_loop(0, NB, step, 0)
        plsc.subcore_barrier()
        a = jnp.minimum(sid * WT, Dh - WT)
        pltpu.sync_copy(den_sh.at[pl.ds(a, WT)], den_out.at[pl.ds(lo + a, WT)])

    return k1


@functools.lru_cache(None)
def _build_k2(D, F, nrels, fwd, H):
    C = F // H
    C16 = C // 16
    if fwd:
        CH = _ru(D, 16)
        NCH = 1
    else:
        CH = (851968 // F) // 128 * 128 - 128
        NCH = -(-D // CH)
    JUNK = CH
    CHr = _ru(CH + 1, 128)
    ZT = CHr // 16
    ZB = min(64, ZT)
    NZ = -(-ZT // ZB)
    WT = _ru(-(-CH // 16), 8)
    if fwd:
        EPW = _EPAD // 32
        NB = EPW // _B
    else:
        EPW = _EPAD // 16
        NB = EPW // _B
    CAP = EPW + _BG

    if fwd:
        out_type = jax.ShapeDtypeStruct((2, CH, F), jnp.float32)
    else:
        out_type = jax.ShapeDtypeStruct((_ru(D, 16), F), jnp.float32)

    @functools.partial(
        pl.kernel,
        out_type=out_type,
        mesh=_mesh(),
        compiler_params=pltpu.CompilerParams(use_tc_tiling_on_sc=False, needs_layout_passes=False),
        scratch_types=[
            pltpu.VMEM((_B,), jnp.int32),
            pltpu.VMEM((_B,), jnp.int32),
            pltpu.VMEM((CAP,), jnp.int32),
            pltpu.VMEM((CAP,), jnp.int32),
            pltpu.VMEM((CAP,), jnp.int32),
            pltpu.VMEM((_BG,), jnp.int32),
            pltpu.VMEM((_BG,), jnp.int32),
            pltpu.VMEM((_BG,), jnp.int32),
            pltpu.VMEM((_BG,), jnp.int32),
            pltpu.VMEM((_BG, F), jnp.float32),
            pltpu.VMEM((_BG, _HP), jnp.float32),
            pltpu.VMEM((_BG, _HP), jnp.float32),
            pltpu.VMEM((ZB, F), jnp.float32),
            pltpu.VMEM_SHARED((CHr, F), jnp.float32),
            pltpu.SemaphoreType.DMA,
            pltpu.SemaphoreType.DMA,
            pltpu.SemaphoreType.DMA,
        ],
    )
    def k2(*args):
        rel_refs = []
        for r in range(nrels):
            rel_refs.append(args[5 * r:5 * r + 5])
        out = args[5 * nrels]
        (src_v, dst_v, srcc, ldstc, eposc, src_bg, ldst_bg, epos_bg, dpos_bg,
         rows, srows, drows, zer, acc, sem1, sem2, sem3) = args[5 * nrels + 1:]
        cid = lax.axis_index("c")
        sid = lax.axis_index("s")

        _zero_fill(zer, ZB, F // 16)
        if fwd:
            ebase = (cid * 16 + sid) * EPW
        else:
            ebase = sid * EPW

        def chunk_body(lo, valid):
            zbase = sid * ZT
            for k in range(NZ):
                st = min(k * ZB, ZT - ZB)
                pltpu.sync_copy(zer, acc.at[pl.ds(zbase + st, ZB)])
            plsc.subcore_barrier()

            for (xs, src, dst, s_in, den) in rel_refs:
                def scan(j, cnt):
                    off = ebase + j * _B
                    pltpu.sync_copy(src.at[pl.ds(off, _B)], src_v)
                    pltpu.sync_copy(dst.at[pl.ds(off, _B)], dst_v)
                    for g in range(_B // 16):
                        dv = dst_v[pl.ds(g * 16, 16)]
                        sv = src_v[pl.ds(g * 16, 16)]
                        m = (dv >= lo) & (dv < lo + CH)
                        ld = dv - lo
                        ep = off + g * 16 + lax.iota(jnp.int32, 16)
                        mi = m.astype(jnp.int32)
                        pos = cnt + plsc.cumsum(mi) - mi
                        plsc.store_scatter(srcc, [pos], sv, mask=m)
                        plsc.store_scatter(ldstc, [pos], ld, mask=m)
                        plsc.store_scatter(eposc, [pos], ep, mask=m)
                        cnt = cnt + jnp.sum(mi)
                    return cnt
                cnt = lax.fori_loop(0, NB, scan, 0)

                zi = jnp.zeros((16,), jnp.int32)
                ji = jnp.full((16,), JUNK, jnp.int32)
                iota16 = lax.iota(jnp.int32, 16)
                for k in range(_BG // 16):
                    pos = cnt + k * 16 + iota16
                    plsc.store_scatter(srcc, [pos], zi)
                    plsc.store_scatter(ldstc, [pos], ji)
                    plsc.store_scatter(eposc, [pos], zi)

                nbat = (cnt + _BG - 1) // _BG

                def proc(i, _):
                    o = i * _BG
                    for k in range(_BG // 16):
                        sv = srcc[pl.ds(o + k * 16, 16)]
                        src_bg[pl.ds(k * 16, 16)] = sv
                        lv = ldstc[pl.ds(o + k * 16, 16)]
                        ldst_bg[pl.ds(k * 16, 16)] = lv
                        dpos_bg[pl.ds(k * 16, 16)] = jnp.minimum(lv + lo, D - 1)
                        ev = eposc[pl.ds(o + k * 16, 16)]
                        epos_bg[pl.ds(k * 16, 16)] = ev
                    g1 = pltpu.async_copy(xs.at[src_bg], rows, sem1)
                    g2 = pltpu.async_copy(s_in.at[epos_bg], srows, sem2)
                    g3 = pltpu.async_copy(den.at[dpos_bg], drows, sem3)
                    g1.wait()
                    g2.wait()
                    g3.wait()

                    def scale(e, _):
                        sv = srows[e, :]
                        dv = drows[e, :]
                        av = sv / (dv + 1e-16)
                        for h in range(H):
                            a_h = av[h]
                            for k2_ in range(C16):
                                col = h * C + k2_ * 16
                                rows[e, pl.ds(col, 16)] = (
                                    rows[e, pl.ds(col, 16)] * a_h)
                        return 0
                    lax.fori_loop(0, _BG, scale, 0)
                    pltpu.sync_copy(rows, acc.at[ldst_bg], add=True)
                    return 0

                lax.fori_loop(0, nbat, proc, 0)

            plsc.subcore_barrier()
            a = jnp.maximum(0, jnp.minimum(sid * WT, valid - WT))
            if fwd:
                pltpu.sync_copy(acc.at[pl.ds(a, WT)],
                                out.at[cid, pl.ds(lo + a, WT)])
            else:
                pltpu.sync_copy(acc.at[pl.ds(a, WT)],
                                out.at[pl.ds(lo + a, WT)])
            plsc.subcore_barrier()

        if fwd:
            chunk_body(0, CH)
        else:
            nch = jnp.where(cid == 0, (NCH + 1) // 2, NCH // 2)

            def cloop(k, _):
                lo = (2 * k + cid) * CH
                valid = jnp.minimum(CH, _ru(D, 16) - lo)
                chunk_body(lo, valid)
                return 0
            lax.fori_loop(0, nch, cloop, 0)

    return k2


def _fold_att(W, att):
    din = W.shape[0]
    H, C = att.shape
    wt = (W.reshape(din, H, C) * att[None]).sum(-1)
    return jnp.pad(wt, ((0, 0), (0, _HP - H)))


def _pad_edges(src, dst):
    ns = jnp.concatenate(
        [src.astype(jnp.int32), jnp.zeros((_EPAD - _E,), jnp.int32)])
    nd = jnp.concatenate(
        [dst.astype(jnp.int32), jnp.full((_EPAD - _E,), _PADDST, jnp.int32)])
    return ns, nd


def _gat_sc(x_src, x_dst, src_p, dst_p, p, heads, out_ch, n_dst, fwd_k2):
    a_s = x_src @ _fold_att(p['W_src'], p['att_src'])
    a_d = x_dst @ _fold_att(p['W_dst'], p['att_dst'])
    s, den = _build_k1(n_dst)(a_s, a_d, src_p, dst_p)
    xs = x_src @ p['W_src']
    k2 = _build_k2(n_dst, heads * out_ch, 1, fwd_k2, heads)
    out = k2(xs, src_p, dst_p, s, den)
    if fwd_k2:
        out = out[0, :n_dst] + out[1, :n_dst]
    return out


def _gat_dual_sc(xa_src, xb_src, x_dst, pa, pb, ea, eb, heads, out_ch, n_dst):
    sa_p, da_p = ea
    sb_p, db_p = eb
    a_sa = xa_src @ _fold_att(pa['W_src'], pa['att_src'])
    a_da = x_dst @ _fold_att(pa['W_dst'], pa['att_dst'])
    a_sb = xb_src @ _fold_att(pb['W_src'], pb['att_src'])
    a_db = x_dst @ _fold_att(pb['W_dst'], pb['att_dst'])
    s_a, den_a = _build_k1(n_dst)(a_sa, a_da, sa_p, da_p)
    s_b, den_b = _build_k1(n_dst)(a_sb, a_db, sb_p, db_p)
    xs_a = xa_src @ pa['W_src']
    xs_b = xb_src @ pb['W_src']
    k2 = _build_k2(n_dst, heads * out_ch, 2, False, heads)
    out = k2(xs_a, sa_p, da_p, s_a, den_a, xs_b, sb_p, db_p, s_b, den_b)
    return out[:n_dst]


def kernel(x_employee, x_department, x_title, src_works_in, dst_works_in,
           src_has_role, dst_has_role, src_rev_works_in, dst_rev_works_in,
           src_rev_has_role, dst_rev_has_role, params):
    x_e, x_d, x_t = x_employee, x_department, x_title
    p = params

    e_wi = _pad_edges(src_works_in, dst_works_in)
    e_hr = _pad_edges(src_has_role, dst_has_role)
    e_rwi = _pad_edges(src_rev_works_in, dst_rev_works_in)
    e_rhr = _pad_edges(src_rev_has_role, dst_rev_has_role)

    agg_d = _gat_sc(x_e, x_d, *e_wi, p['c1_wi'], _HEADS, _HID, _ND, True)
    agg_t = _gat_sc(x_e, x_t, *e_hr, p['c1_hr'], _HEADS, _HID, _NT, True)
    agg_e = _gat_dual_sc(x_d, x_t, x_e, p['c1_rwi'], p['c1_rhr'],
                         e_rwi, e_rhr, _HEADS, _HID, _NE)
    h_d = jax.nn.relu(agg_d + p['c1_wi']['bias'])
    h_t = jax.nn.relu(agg_t + p['c1_hr']['bias'])
    h_e = jax.nn.relu(agg_e + p['c1_rwi']['bias'] + p['c1_rhr']['bias'])

    agg2_e = _gat_dual_sc(h_d, h_t, h_e, p['c2_rwi'], p['c2_rhr'],
                          e_rwi, e_rhr, 1, _HID, _NE)
    g_e = jax.nn.relu(agg2_e + p['c2_rwi']['bias'] + p['c2_rhr']['bias'])

    out = g_e @ p['lin_W'] + p['lin_b']
    return jax.nn.log_softmax(out, axis=1)


# speedup vs baseline: 12.7627x; 1.5092x over previous
import functools

import jax
import jax.numpy as jnp
from jax import lax
from jax.experimental import pallas as pl
from jax.experimental.pallas import tpu as pltpu
from jax.experimental.pallas import tpu_sc as plsc

_NE, _ND, _NT = 100000, 500, 2000
_E = 100000
_DIN = 128
_HID = 64
_HEADS = 4
_DOUT = 16
_HP = 16
_EPAD = 106496
_B = 256
_BG = 64
_SMALLBITS = 11
_SMALLPAD = 2047
_BIGPAD = 131071
_SPMEM_WORDS = 2097151


def _ru(x, m):
    return -(-x // m) * m


def _mesh():
    return plsc.VectorSubcoreMesh(core_axis_name="c", subcore_axis_name="s")


def _zero_fill(buf, rows, width16):
    z = jnp.zeros((16,), jnp.float32)

    def zb(r, _):
        for k in range(width16):
            buf[r, pl.ds(k * 16, 16)] = z
        return 0
    lax.fori_loop(0, rows, zb, 0)


def _zero_shared(zer, zrows, acc, base, zt, sem):
    nz = -(-zt // zrows)
    cps = []
    for k in range(nz):
        st = min(k * zrows, zt - zrows)
        cps.append(pltpu.async_copy(zer.at[pl.ds(0, zrows)],
                                    acc.at[pl.ds(base + st, zrows)], sem))
    for cp in cps:
        cp.wait()


def _unpack(w, big_is_dst):
    small = w & _SMALLPAD
    big = w >> _SMALLBITS
    if big_is_dst:
        return small, big
    return big, small


@functools.lru_cache(None)
def _build_k1(D):
    big_dst = D > 2048
    Dh = _ru(D, 16) // 2
    JUNK = Dh
    DLr = _ru(Dh + 1, 128)
    ZT = DLr // 16
    ZB = min(128, ZT)
    EPT = _EPAD // 16
    NB = EPT // _B
    WT = _ru(-(-Dh // 16), 8)

    @functools.partial(
        pl.kernel,
        out_type=(jax.ShapeDtypeStruct((_EPAD, _HP), jnp.float32),
                  jax.ShapeDtypeStruct((2 * Dh, _HP), jnp.float32)),
        mesh=_mesh(),
        compiler_params=pltpu.CompilerParams(
            use_tc_tiling_on_sc=False, needs_layout_passes=False),
        scratch_types=[
            pltpu.VMEM((EPT,), jnp.int32),
            pltpu.VMEM((2, 128), jnp.int32),
            pltpu.VMEM((2, 128), jnp.int32),
            pltpu.VMEM((2, 128), jnp.int32),
            pltpu.VMEM((_B, _HP), jnp.float32),
            pltpu.VMEM((_B, _HP), jnp.float32),
            pltpu.VMEM((_B, _HP), jnp.float32),
            pltpu.VMEM((128, _HP), jnp.float32),
            pltpu.VMEM_SHARED((DLr, _HP), jnp.float32),
            pltpu.SemaphoreType.DMA,
            pltpu.SemaphoreType.DMA,
        ],
    )
    def k1(a_s, a_d, epk_in, s_out, den_out,
           epk, src2, dstc2, ldst2, as_r, ad_r, s_r, zer, den_sh,
           sem1, sem2):
        cid = lax.axis_index("c")
        sid = lax.axis_index("s")
        lo = cid * Dh

        pltpu.sync_copy(epk_in.at[pl.ds(sid * EPT, EPT)], epk)
        _zero_fill(zer, min(128, ZT), 1)
        _zero_shared(zer, min(128, ZT), den_sh, sid * ZT, ZT, sem1)
        plsc.subcore_barrier()

        def step(j, carry):
            off = sid * EPT + j * _B
            loff = j * _B
            for g in range(_B // 16):
                w = epk[pl.ds(loff + g * 16, 16)]
                sv, dv = _unpack(w, big_dst)
                dc = jnp.minimum(dv, D - 1)
                inh = (dv >= lo) & (dv < lo + Dh)
                ld = jnp.where(inh, dv - lo, JUNK)
                src2[g // 8, pl.ds((g % 8) * 16, 16)] = sv
                dstc2[g // 8, pl.ds((g % 8) * 16, 16)] = dc
                ldst2[g // 8, pl.ds((g % 8) * 16, 16)] = ld
            cps = []
            for k in range(2):
                cps.append(pltpu.async_copy(
                    a_s.at[src2.at[k]], as_r.at[pl.ds(k * 128, 128)], sem1))
                cps.append(pltpu.async_copy(
                    a_d.at[dstc2.at[k]], ad_r.at[pl.ds(k * 128, 128)], sem2))
            for cp in cps:
                cp.wait()

            def ebody(e4, _):
                for u in range(4):
                    e = e4 * 4 + u
                    x = as_r[e, :] + ad_r[e, :]
                    x = jnp.maximum(x, 0.2 * x)
                    s_r[e, :] = jnp.exp(x)
                return 0
            lax.fori_loop(0, _B // 4, ebody, 0)

            for k in range(2):
                pltpu.sync_copy(s_r.at[pl.ds(k * 128, 128)],
                                den_sh.at[ldst2.at[k]], add=True)

            @pl.when(cid == 0)
            def _():
                pltpu.sync_copy(s_r, s_out.at[pl.ds(off, _B)])
            return carry

        lax.fori_loop(0, NB, step, 0)
        plsc.subcore_barrier()
        a = jnp.minimum(sid * WT, Dh - WT)
        pltpu.sync_copy(den_sh.at[pl.ds(a, WT)], den_out.at[pl.ds(lo + a, WT)])

    return k1


@functools.lru_cache(None)
def _build_k2(D, F, nrels, fwd, H):
    C = F // H
    C16 = C // 16
    if fwd:
        EPW = _EPAD // 32
    else:
        EPW = _EPAD // 16
    NB = EPW // _B
    CAP = EPW + _BG
    ZR = 32

    tile_words = (nrels * EPW + 3 * CAP + 4 * _BG + _BG * F + 2 * _BG * _HP
                  + ZR * F)
    budget = _SPMEM_WORDS - 16 * tile_words - 16384
    if fwd:
        CH = _ru(D, 16)
        NCH = 1
    else:
        CH = (budget // F) // 128 * 128 - 128
        NCH = -(-D // CH)
    JUNK = CH
    CHr = _ru(CH + 1, 128)
    assert fwd or CHr * F <= budget, (CHr, F, budget)
    ZT = CHr // 16
    WT = _ru(-(-CH // 16), 8)

    if fwd:
        out_type = jax.ShapeDtypeStruct((2, CH, F), jnp.float32)
    else:
        out_type = jax.ShapeDtypeStruct((_ru(D, 16), F), jnp.float32)

    @functools.partial(
        pl.kernel,
        out_type=out_type,
        mesh=_mesh(),
        compiler_params=pltpu.CompilerParams(
            use_tc_tiling_on_sc=False, needs_layout_passes=False),
        scratch_types=[
            pltpu.VMEM((nrels, EPW), jnp.int32),
            pltpu.VMEM((CAP,), jnp.int32),
            pltpu.VMEM((CAP,), jnp.int32),
            pltpu.VMEM((CAP,), jnp.int32),
            pltpu.VMEM((_BG,), jnp.int32),
            pltpu.VMEM((_BG,), jnp.int32),
            pltpu.VMEM((_BG,), jnp.int32),
            pltpu.VMEM((_BG,), jnp.int32),
            pltpu.VMEM((_BG, F), jnp.float32),
            pltpu.VMEM((_BG, _HP), jnp.float32),
            pltpu.VMEM((_BG, _HP), jnp.float32),
            pltpu.VMEM((ZR, F), jnp.float32),
            pltpu.VMEM_SHARED((CHr, F), jnp.float32),
            pltpu.SemaphoreType.DMA,
            pltpu.SemaphoreType.DMA,
            pltpu.SemaphoreType.DMA,
        ],
    )
    def k2(*args):
        rel_refs = []
        for r in range(nrels):
            rel_refs.append(args[4 * r:4 * r + 4])
        out = args[4 * nrels]
        (epk, srcc, ldstc, eposc, src_bg, ldst_bg, epos_bg, dpos_bg,
         rows, srows, drows, zer, acc, sem1, sem2, sem3) = args[4 * nrels + 1:]
        cid = lax.axis_index("c")
        sid = lax.axis_index("s")

        _zero_fill(zer, ZR, F // 16)
        if fwd:
            ebase = (cid * 16 + sid) * EPW
        else:
            ebase = sid * EPW
        for r in range(nrels):
            pltpu.sync_copy(rel_refs[r][1].at[pl.ds(ebase, EPW)], epk.at[r])

        def chunk_body(lo, valid):
            _zero_shared(zer, ZR, acc, sid * ZT, ZT, sem1)
            plsc.subcore_barrier()

            for r, (xs, ew, s_in, den) in enumerate(rel_refs):
                def scan(j, cnt):
                    off = ebase + j * _B
                    loff = j * _B
                    for g in range(_B // 16):
                        w = epk[r, pl.ds(loff + g * 16, 16)]
                        sv, dv = _unpack(w, not fwd)
                        m = (dv >= lo) & (dv < lo + CH)
                        ld = dv - lo
                        ep = off + g * 16 + lax.iota(jnp.int32, 16)
                        mi = m.astype(jnp.int32)
                        pos = cnt + plsc.cumsum(mi) - mi
                        plsc.store_scatter(srcc, [pos], sv, mask=m)
                        plsc.store_scatter(ldstc, [pos], ld, mask=m)
                        plsc.store_scatter(eposc, [pos], ep, mask=m)
                        cnt = cnt + jnp.sum(mi)
                    return cnt
                cnt = lax.fori_loop(0, NB, scan, 0)

                zi = jnp.zeros((16,), jnp.int32)
                ji = jnp.full((16,), JUNK, jnp.int32)
                iota16 = lax.iota(jnp.int32, 16)
                for k in range(_BG // 16):
                    pos = cnt + k * 16 + iota16
                    plsc.store_scatter(srcc, [pos], zi)
                    plsc.store_scatter(ldstc, [pos], ji)
                    plsc.store_scatter(eposc, [pos], zi)

                nbat = (cnt + _BG - 1) // _BG

                def proc(i, _):
                    o = i * _BG
                    for k in range(_BG // 16):
                        sv = srcc[pl.ds(o + k * 16, 16)]
                        src_bg[pl.ds(k * 16, 16)] = sv
                        lv = ldstc[pl.ds(o + k * 16, 16)]
                        ldst_bg[pl.ds(k * 16, 16)] = lv
                        dpos_bg[pl.ds(k * 16, 16)] = jnp.minimum(lv + lo, D - 1)
                        ev = eposc[pl.ds(o + k * 16, 16)]
                        epos_bg[pl.ds(k * 16, 16)] = ev
                    g1 = pltpu.async_copy(xs.at[src_bg], rows, sem1)
                    g2 = pltpu.async_copy(s_in.at[epos_bg], srows, sem2)
                    g3 = pltpu.async_copy(den.at[dpos_bg], drows, sem3)
                    g1.wait()
                    g2.wait()
                    g3.wait()

                    def scale(e, _):
                        sv2 = srows[e, :]
                        dv2 = drows[e, :]
                        av = sv2 / (dv2 + 1e-16)
                        for h in range(H):
                            a_h = av[h]
                            for k2_ in range(C16):
                                col = h * C + k2_ * 16
                                rows[e, pl.ds(col, 16)] = (
                                    rows[e, pl.ds(col, 16)] * a_h)
                        return 0
                    lax.fori_loop(0, _BG, scale, 0)
                    pltpu.sync_copy(rows, acc.at[ldst_bg], add=True)
                    return 0

                lax.fori_loop(0, nbat, proc, 0)

            plsc.subcore_barrier()
            a = jnp.maximum(0, jnp.minimum(sid * WT, valid - WT))
            if fwd:
                pltpu.sync_copy(acc.at[pl.ds(a, WT)],
                                out.at[cid, pl.ds(lo + a, WT)])
            else:
                pltpu.sync_copy(acc.at[pl.ds(a, WT)],
                                out.at[pl.ds(lo + a, WT)])
            plsc.subcore_barrier()

        if fwd:
            chunk_body(0, CH)
        else:
            nch = jnp.where(cid == 0, (NCH + 1) // 2, NCH // 2)

            def cloop(k, _):
                lo = (2 * k + cid) * CH
                valid = jnp.minimum(CH, _ru(D, 16) - lo)
                chunk_body(lo, valid)
                return 0
            lax.fori_loop(0, nch, cloop, 0)

    return k2


def _fold_att(W, att):
    din = W.shape[0]
    H, C = att.shape
    wt = (W.reshape(din, H, C) * att[None]).sum(-1)
    return jnp.pad(wt, ((0, 0), (0, _HP - H)))


def _pack_edges(src, dst, big_is_dst):
    src = src.astype(jnp.int32)
    dst = dst.astype(jnp.int32)
    if big_is_dst:
        w = (dst << _SMALLBITS) | src
        pad = _BIGPAD << _SMALLBITS
    else:
        w = (src << _SMALLBITS) | dst
        pad = _SMALLPAD
    return jnp.concatenate([w, jnp.full((_EPAD - _E,), pad, jnp.int32)])


def _gat_sc(x_src, x_dst, epk, p, heads, out_ch, n_dst, fwd_k2):
    a_s = x_src @ _fold_att(p['W_src'], p['att_src'])
    a_d = x_dst @ _fold_att(p['W_dst'], p['att_dst'])
    s, den = _build_k1(n_dst)(a_s, a_d, epk)
    xs = x_src @ p['W_src']
    k2 = _build_k2(n_dst, heads * out_ch, 1, fwd_k2, heads)
    out = k2(xs, epk, s, den)
    if fwd_k2:
        out = out[0, :n_dst] + out[1, :n_dst]
    return out


def _gat_dual_sc(xa_src, xb_src, x_dst, pa, pb, epa, epb, heads, out_ch, n_dst):
    a_sa = xa_src @ _fold_att(pa['W_src'], pa['att_src'])
    a_da = x_dst @ _fold_att(pa['W_dst'], pa['att_dst'])
    a_sb = xb_src @ _fold_att(pb['W_src'], pb['att_src'])
    a_db = x_dst @ _fold_att(pb['W_dst'], pb['att_dst'])
    s_a, den_a = _build_k1(n_dst)(a_sa, a_da, epa)
    s_b, den_b = _build_k1(n_dst)(a_sb, a_db, epb)
    xs_a = xa_src @ pa['W_src']
    xs_b = xb_src @ pb['W_src']
    k2 = _build_k2(n_dst, heads * out_ch, 2, False, heads)
    out = k2(xs_a, epa, s_a, den_a, xs_b, epb, s_b, den_b)
    return out[:n_dst]


def kernel(x_employee, x_department, x_title, src_works_in, dst_works_in,
           src_has_role, dst_has_role, src_rev_works_in, dst_rev_works_in,
           src_rev_has_role, dst_rev_has_role, params):
    x_e, x_d, x_t = x_employee, x_department, x_title
    p = params

    e_wi = _pack_edges(src_works_in, dst_works_in, False)
    e_hr = _pack_edges(src_has_role, dst_has_role, False)
    e_rwi = _pack_edges(src_rev_works_in, dst_rev_works_in, True)
    e_rhr = _pack_edges(src_rev_has_role, dst_rev_has_role, True)

    agg_d = _gat_sc(x_e, x_d, e_wi, p['c1_wi'], _HEADS, _HID, _ND, True)
    agg_t = _gat_sc(x_e, x_t, e_hr, p['c1_hr'], _HEADS, _HID, _NT, True)
    agg_e = _gat_dual_sc(x_d, x_t, x_e, p['c1_rwi'], p['c1_rhr'],
                         e_rwi, e_rhr, _HEADS, _HID, _NE)
    h_d = jax.nn.relu(agg_d + p['c1_wi']['bias'])
    h_t = jax.nn.relu(agg_t + p['c1_hr']['bias'])
    h_e = jax.nn.relu(agg_e + p['c1_rwi']['bias'] + p['c1_rhr']['bias'])

    agg2_e = _gat_dual_sc(h_d, h_t, h_e, p['c2_rwi'], p['c2_rhr'],
                          e_rwi, e_rhr, 1, _HID, _NE)
    g_e = jax.nn.relu(agg2_e + p['c2_rwi']['bias'] + p['c2_rhr']['bias'])

    out = g_e @ p['lin_W'] + p['lin_b']
    return jax.nn.log_softmax(out, axis=1)
